# Initial kernel scaffold; baseline (speedup 1.0000x reference)
import functools

import jax
import jax.numpy as jnp
from jax import lax
from jax.experimental import pallas as pl
from jax.experimental.pallas import tpu as pltpu
from jax.experimental.pallas import tpu_sc as plsc

N = 10000
D1 = 128
D2 = 64
E = 320000
NC = 2
NS = 16
NW = NC * NS
CH = 128
NCHUNK = 79
EPAD = NW * CH * NCHUNK
R = 10240
RPT = R // NS

_mesh = plsc.VectorSubcoreMesh(core_axis_name="c", subcore_axis_name="s")


def _zero_f32(ref, rows, width):
    z = jnp.zeros((16,), jnp.float32)
    for i in range(rows):
        for k in range(width // 16):
            ref[i, pl.ds(k * 16, 16)] = z


def _deg_body(dst_hbm, deg_out, idx_v, ones_v, zb, deg_sh):
    c = lax.axis_index("c")
    s = lax.axis_index("s")
    wid = c * NS + s

    pltpu.sync_copy(dst_hbm.at[wid], idx_v)
    one = jnp.ones((16,), jnp.float32)
    zero = jnp.zeros((16,), jnp.float32)
    for k in range(CH // 16):
        ones_v[pl.ds(k * 16, 16)] = one
    for k in range(RPT // 16):
        zb[pl.ds(k * 16, 16)] = zero

    row0 = s * RPT
    pltpu.sync_copy(zb, deg_sh.at[pl.ds(row0, RPT)])
    plsc.subcore_barrier()

    def body(j, carry):
        pltpu.sync_copy(ones_v, deg_sh.at[idx_v.at[j]], add=True)
        return carry

    lax.fori_loop(0, NCHUNK, body, 0)
    plsc.subcore_barrier()

    pltpu.sync_copy(deg_sh.at[pl.ds(row0, RPT)], zb)
    pltpu.sync_copy(zb, deg_out.at[c, pl.ds(row0, RPT)])


_deg_call = functools.partial(
    pl.kernel,
    out_type=jax.ShapeDtypeStruct((NC, R), jnp.float32),
    mesh=_mesh,
    scratch_types=[
        pltpu.VMEM((NCHUNK, CH), jnp.int32),
        pltpu.VMEM((CH,), jnp.float32),
        pltpu.VMEM((RPT,), jnp.float32),
        pltpu.VMEM_SHARED((R,), jnp.float32),
    ],
)(_deg_body)


def _agg_body(D, g_hbm, src_hbm, dst_hbm, acc_out, sidx, didx, rows, zb,
              acc_sh, sem):
    c = lax.axis_index("c")
    s = lax.axis_index("s")
    wid = c * NS + s

    pltpu.sync_copy(src_hbm.at[wid], sidx)
    pltpu.sync_copy(dst_hbm.at[wid], didx)
    _zero_f32(zb, 16, D)

    row0 = s * RPT
    def zbody(k, carry):
        pltpu.sync_copy(zb, acc_sh.at[pl.ds(row0 + k * 16, 16)])
        return carry
    lax.fori_loop(0, RPT // 16, zbody, 0)
    plsc.subcore_barrier()

    def body(j, carry):
        pltpu.async_copy(g_hbm.at[sidx.at[j]], rows, sem).wait()
        pltpu.sync_copy(rows, acc_sh.at[didx.at[j]], add=True)
        return carry

    lax.fori_loop(0, NCHUNK, body, 0)
    plsc.subcore_barrier()

    def obody(k, carry):
        pltpu.sync_copy(acc_sh.at[pl.ds(row0 + k * CH, CH)], rows)
        pltpu.sync_copy(rows, acc_out.at[c, pl.ds(row0 + k * CH, CH)])
        return carry
    lax.fori_loop(0, RPT // CH, obody, 0)


def _make_agg(D):
    return functools.partial(
        pl.kernel,
        out_type=jax.ShapeDtypeStruct((NC, R, D), jnp.float32),
        mesh=_mesh,
        scratch_types=[
            pltpu.VMEM((NCHUNK, CH), jnp.int32),
            pltpu.VMEM((NCHUNK, CH), jnp.int32),
            pltpu.VMEM((CH, D), jnp.float32),
            pltpu.VMEM((16, D), jnp.float32),
            pltpu.VMEM_SHARED((R, D), jnp.float32),
            pltpu.SemaphoreType.DMA,
        ],
    )(functools.partial(_agg_body, D))


_agg128 = _make_agg(D1)
_agg64 = _make_agg(D2)


def _t0_body(deg_ref, dis_ref):
    deg = deg_ref[0:1, :] + deg_ref[1:2, :] + 1.0
    dis_ref[...] = lax.rsqrt(deg)


def _t0(deg2):
    return pl.pallas_call(
        _t0_body,
        out_shape=jax.ShapeDtypeStruct((1, R), jnp.float32),
    )(deg2)


_BR = 1000


def _t1_body(x_ref, w_ref, dis_ref, o_ref):
    o_ref[...] = jnp.dot(x_ref[...], w_ref[...],
                         preferred_element_type=jnp.float32) * dis_ref[...]


def _t1(x, W1, dis_col):
    return pl.pallas_call(
        _t1_body,
        grid=(N // _BR,),
        in_specs=[
            pl.BlockSpec((_BR, D1), lambda i: (i, 0)),
            pl.BlockSpec((D1, D1), lambda i: (0, 0)),
            pl.BlockSpec((_BR, 1), lambda i: (i, 0)),
        ],
        out_specs=pl.BlockSpec((_BR, D1), lambda i: (i, 0)),
        out_shape=jax.ShapeDtypeStruct((N, D1), jnp.float32),
    )(x, W1, dis_col)


def _t2_body(p_ref, g_ref, dis_ref, b_ref, w_ref, o_ref):
    dis = dis_ref[...]
    z = dis * (p_ref[0] + p_ref[1] + g_ref[...]) + b_ref[...]
    h = jnp.maximum(z, 0.0)
    o_ref[...] = jnp.dot(h, w_ref[...],
                         preferred_element_type=jnp.float32) * dis


def _t2(p1, g1, dis_col, b1, W2):
    return pl.pallas_call(
        _t2_body,
        grid=(N // _BR,),
        in_specs=[
            pl.BlockSpec((NC, _BR, D1), lambda i: (0, i, 0)),
            pl.BlockSpec((_BR, D1), lambda i: (i, 0)),
            pl.BlockSpec((_BR, 1), lambda i: (i, 0)),
            pl.BlockSpec((1, D1), lambda i: (0, 0)),
            pl.BlockSpec((D1, D2), lambda i: (0, 0)),
        ],
        out_specs=pl.BlockSpec((_BR, D2), lambda i: (i, 0)),
        out_shape=jax.ShapeDtypeStruct((N, D2), jnp.float32),
    )(p1, g1, dis_col, b1.reshape(1, D1), W2)


def _t3_body(p_ref, g_ref, dis_ref, b_ref, o_ref):
    z = dis_ref[...] * (p_ref[0] + p_ref[1] + g_ref[...]) + b_ref[...]
    m = jnp.max(z, axis=-1, keepdims=True)
    zs = z - m
    o_ref[...] = zs - jnp.log(jnp.sum(jnp.exp(zs), axis=-1, keepdims=True))


def _t3(p2, g2, dis_col, b2):
    return pl.pallas_call(
        _t3_body,
        grid=(N // _BR,),
        in_specs=[
            pl.BlockSpec((NC, _BR, D2), lambda i: (0, i, 0)),
            pl.BlockSpec((_BR, D2), lambda i: (i, 0)),
            pl.BlockSpec((_BR, 1), lambda i: (i, 0)),
            pl.BlockSpec((1, D2), lambda i: (0, 0)),
        ],
        out_specs=pl.BlockSpec((_BR, D2), lambda i: (i, 0)),
        out_shape=jax.ShapeDtypeStruct((N, D2), jnp.float32),
    )(p2, g2, dis_col, b2.reshape(1, D2))


def kernel(x, edge_index, W1, b1, W2, b2):
    src = edge_index[0].astype(jnp.int32)
    dst = edge_index[1].astype(jnp.int32)
    pad = EPAD - E
    src_p = jnp.concatenate([src, jnp.zeros((pad,), jnp.int32)])
    dst_p = jnp.concatenate([dst, jnp.full((pad,), N, jnp.int32)])
    src_p = src_p.reshape(NW, NCHUNK, CH)
    dst_p = dst_p.reshape(NW, NCHUNK, CH)

    deg2 = _deg_call(dst_p)
    dis_col = _t0(deg2).reshape(R, 1)
    g1 = _t1(x, W1, dis_col[:N])
    p1 = _agg128(g1, src_p, dst_p)
    g2 = _t2(p1, g1, dis_col[:N], b1, W2)
    p2 = _agg64(g2, src_p, dst_p)
    return _t3(p2, g2, dis_col[:N], b2)


# speedup vs baseline: 15.6819x; 15.6819x over previous
import functools

import jax
import jax.numpy as jnp
from jax import lax
from jax.experimental import pallas as pl
from jax.experimental.pallas import tpu as pltpu
from jax.experimental.pallas import tpu_sc as plsc

N = 10000
D1 = 128
D2 = 64
E = 320000
NC = 2
NS = 16
NW = NC * NS
CH = 128
NCHUNK = 79
EPAD = NW * CH * NCHUNK
R = 10240
RPT = R // NS

_mesh = plsc.VectorSubcoreMesh(core_axis_name="c", subcore_axis_name="s")


def _zero_f32(ref, rows, width):
    z = jnp.zeros((16,), jnp.float32)
    for i in range(rows):
        for k in range(width // 16):
            ref[i, pl.ds(k * 16, 16)] = z


def _deg_body(dst_hbm, deg_out, idx_v, ones_v, zb, deg_sh):
    c = lax.axis_index("c")
    s = lax.axis_index("s")
    wid = c * NS + s

    pltpu.sync_copy(dst_hbm.at[wid], idx_v)
    one = jnp.ones((16,), jnp.float32)
    zero = jnp.zeros((16,), jnp.float32)
    for k in range(CH // 16):
        ones_v[pl.ds(k * 16, 16)] = one
    for k in range(RPT // 16):
        zb[pl.ds(k * 16, 16)] = zero

    row0 = s * RPT
    pltpu.sync_copy(zb, deg_sh.at[pl.ds(row0, RPT)])
    plsc.subcore_barrier()

    def body(j, carry):
        pltpu.sync_copy(ones_v, deg_sh.at[idx_v.at[j]], add=True)
        return carry

    lax.fori_loop(0, NCHUNK, body, 0)
    plsc.subcore_barrier()

    pltpu.sync_copy(deg_sh.at[pl.ds(row0, RPT)], zb)
    pltpu.sync_copy(zb, deg_out.at[c, pl.ds(row0, RPT)])


_deg_call = functools.partial(
    pl.kernel,
    out_type=jax.ShapeDtypeStruct((NC, R), jnp.float32),
    mesh=_mesh,
    scratch_types=[
        pltpu.VMEM((NCHUNK, CH), jnp.int32),
        pltpu.VMEM((CH,), jnp.float32),
        pltpu.VMEM((RPT,), jnp.float32),
        pltpu.VMEM_SHARED((R,), jnp.float32),
    ],
)(_deg_body)


def _agg_body(D, g_hbm, src_hbm, dst_hbm, acc_out, sidx, didx, rows, zb,
              acc_sh, sem):
    c = lax.axis_index("c")
    s = lax.axis_index("s")
    wid = c * NS + s

    pltpu.sync_copy(src_hbm.at[wid], sidx)
    pltpu.sync_copy(dst_hbm.at[wid], didx)
    _zero_f32(zb, 16, D)

    row0 = s * RPT
    def zbody(k, carry):
        pltpu.sync_copy(zb, acc_sh.at[pl.ds(row0 + k * 16, 16)])
        return carry
    lax.fori_loop(0, RPT // 16, zbody, 0)
    plsc.subcore_barrier()

    def body(j, carry):
        pltpu.async_copy(g_hbm.at[sidx.at[j]], rows, sem).wait()
        pltpu.sync_copy(rows, acc_sh.at[didx.at[j]], add=True)
        return carry

    lax.fori_loop(0, NCHUNK, body, 0)
    plsc.subcore_barrier()

    def obody(k, carry):
        pltpu.sync_copy(acc_sh.at[pl.ds(row0 + k * CH, CH)], rows)
        pltpu.sync_copy(rows, acc_out.at[c, pl.ds(row0 + k * CH, CH)])
        return carry
    lax.fori_loop(0, RPT // CH, obody, 0)


def _make_agg(D):
    return functools.partial(
        pl.kernel,
        out_type=jax.ShapeDtypeStruct((NC, R, D), jnp.float32),
        mesh=_mesh,
        scratch_types=[
            pltpu.VMEM((NCHUNK, CH), jnp.int32),
            pltpu.VMEM((NCHUNK, CH), jnp.int32),
            pltpu.VMEM((CH, D), jnp.float32),
            pltpu.VMEM((16, D), jnp.float32),
            pltpu.VMEM_SHARED((R, D), jnp.float32),
            pltpu.SemaphoreType.DMA,
        ],
        compiler_params=pltpu.CompilerParams(use_tc_tiling_on_sc=False),
    )(functools.partial(_agg_body, D))


_agg128 = _make_agg(D1)
_agg64 = _make_agg(D2)


def _t0_body(deg_ref, dis_ref):
    deg = deg_ref[0:1, :] + deg_ref[1:2, :] + 1.0
    dis_ref[...] = lax.rsqrt(deg)


def _t0(deg2):
    return pl.pallas_call(
        _t0_body,
        out_shape=jax.ShapeDtypeStruct((1, R), jnp.float32),
    )(deg2)


_BR = 1000


def _t1_body(x_ref, w_ref, dis_ref, o_ref):
    o_ref[...] = jnp.dot(x_ref[...], w_ref[...],
                         preferred_element_type=jnp.float32) * dis_ref[...]


def _t1(x, W1, dis_col):
    return pl.pallas_call(
        _t1_body,
        grid=(N // _BR,),
        in_specs=[
            pl.BlockSpec((_BR, D1), lambda i: (i, 0)),
            pl.BlockSpec((D1, D1), lambda i: (0, 0)),
            pl.BlockSpec((_BR, 1), lambda i: (i, 0)),
        ],
        out_specs=pl.BlockSpec((_BR, D1), lambda i: (i, 0)),
        out_shape=jax.ShapeDtypeStruct((N, D1), jnp.float32),
    )(x, W1, dis_col)


def _t2_body(p_ref, g_ref, dis_ref, b_ref, w_ref, o_ref):
    dis = dis_ref[...]
    z = dis * (p_ref[0] + p_ref[1] + g_ref[...]) + b_ref[...]
    h = jnp.maximum(z, 0.0)
    o_ref[...] = jnp.dot(h, w_ref[...],
                         preferred_element_type=jnp.float32) * dis


def _t2(p1, g1, dis_col, b1, W2):
    return pl.pallas_call(
        _t2_body,
        grid=(N // _BR,),
        in_specs=[
            pl.BlockSpec((NC, _BR, D1), lambda i: (0, i, 0)),
            pl.BlockSpec((_BR, D1), lambda i: (i, 0)),
            pl.BlockSpec((_BR, 1), lambda i: (i, 0)),
            pl.BlockSpec((1, D1), lambda i: (0, 0)),
            pl.BlockSpec((D1, D2), lambda i: (0, 0)),
        ],
        out_specs=pl.BlockSpec((_BR, D2), lambda i: (i, 0)),
        out_shape=jax.ShapeDtypeStruct((N, D2), jnp.float32),
    )(p1, g1, dis_col, b1.reshape(1, D1), W2)


def _t3_body(p_ref, g_ref, dis_ref, b_ref, o_ref):
    z = dis_ref[...] * (p_ref[0] + p_ref[1] + g_ref[...]) + b_ref[...]
    m = jnp.max(z, axis=-1, keepdims=True)
    zs = z - m
    o_ref[...] = zs - jnp.log(jnp.sum(jnp.exp(zs), axis=-1, keepdims=True))


def _t3(p2, g2, dis_col, b2):
    return pl.pallas_call(
        _t3_body,
        grid=(N // _BR,),
        in_specs=[
            pl.BlockSpec((NC, _BR, D2), lambda i: (0, i, 0)),
            pl.BlockSpec((_BR, D2), lambda i: (i, 0)),
            pl.BlockSpec((_BR, 1), lambda i: (i, 0)),
            pl.BlockSpec((1, D2), lambda i: (0, 0)),
        ],
        out_specs=pl.BlockSpec((_BR, D2), lambda i: (i, 0)),
        out_shape=jax.ShapeDtypeStruct((N, D2), jnp.float32),
    )(p2, g2, dis_col, b2.reshape(1, D2))


def kernel(x, edge_index, W1, b1, W2, b2):
    src = edge_index[0].astype(jnp.int32)
    dst = edge_index[1].astype(jnp.int32)
    pad = EPAD - E
    src_p = jnp.concatenate([src, jnp.zeros((pad,), jnp.int32)])
    dst_p = jnp.concatenate([dst, jnp.full((pad,), N, jnp.int32)])
    src_p = src_p.reshape(NW, NCHUNK, CH)
    dst_p = dst_p.reshape(NW, NCHUNK, CH)

    deg2 = _deg_call(dst_p)
    dis_col = _t0(deg2).reshape(R, 1)
    g1 = _t1(x, W1, dis_col[:N])
    p1 = _agg128(g1, src_p, dst_p)
    g2 = _t2(p1, g1, dis_col[:N], b1, W2)
    p2 = _agg64(g2, src_p, dst_p)
    return _t3(p2, g2, dis_col[:N], b2)
